# baseline (speedup 1.0000x reference)
import functools
import math

import jax
import jax.numpy as jnp
from jax.experimental import pallas as pl

NODE_DIM = 256
FT_DIM = 256
HIDDEN_DIM = 256
TIME_EMB_DIM = 128
T_STEPS = 8
TILE = 400
MAX_POS = 10000.0


def _fold_weights_kernel(atom_ref, w1a_ref, w1f_ref, w2h_ref, b1_ref, b2_ref,
                         a2_ref, ct_ref, bc_ref):
    a1 = jax.lax.dot_general(atom_ref[...], w1a_ref[...],
                             (((1,), (1,)), ((), ())),
                             preferred_element_type=jnp.float32)
    a2_ref[...] = jax.lax.dot_general(a1, w2h_ref[...],
                                      (((1,), (1,)), ((), ())),
                                      preferred_element_type=jnp.float32)
    ct_ref[...] = jnp.dot(w2h_ref[...], w1f_ref[...],
                          preferred_element_type=jnp.float32)
    bc_ref[...] = jax.lax.dot_general(b1_ref[...], w2h_ref[...],
                                      (((1,), (1,)), ((), ())),
                                      preferred_element_type=jnp.float32) + b2_ref[...]


def _main_kernel(h_ref, dt_ref, f_ref, a2_ref, ct_ref, w2t_ref, bc_ref,
                 out_ref):
    tile = f_ref.shape[0]
    hv = h_ref[:, 0]
    lane = jax.lax.broadcasted_iota(jnp.int32, (tile, 128), 1)
    onehot = (lane == hv[:, None]).astype(jnp.float32)
    o = jnp.dot(onehot, a2_ref[...], preferred_element_type=jnp.float32)
    o += jax.lax.dot_general(f_ref[...], ct_ref[...],
                             (((1,), (1,)), ((), ())),
                             preferred_element_type=jnp.float32)
    half = TIME_EMB_DIM // 2
    coef = -math.log(MAX_POS) / (half - 1)
    j = jax.lax.broadcasted_iota(jnp.int32, (tile, half), 1).astype(jnp.float32)
    freq = jnp.exp(j * coef)
    args = dt_ref[:, 0].astype(jnp.float32)[:, None] * freq
    temb = jnp.concatenate([jnp.sin(args), jnp.cos(args)], axis=1)
    o += jax.lax.dot_general(temb, w2t_ref[...],
                             (((1,), (1,)), ((), ())),
                             preferred_element_type=jnp.float32)
    o += bc_ref[...]
    out_ref[...] = jnp.broadcast_to(
        o[:, :, None], (tile, HIDDEN_DIM, T_STEPS)).reshape(tile,
                                                            HIDDEN_DIM * T_STEPS)


@jax.jit
def kernel(diffusion_t, x, h, f, edge_index, edge_attr, batch, atom_emb,
           emb_lin_W, emb_lin_b, edge_emb_table, input_lin_W, input_lin_b,
           cond_emb_table):
    bn = f.shape[0]
    atom_p = jnp.zeros((128, NODE_DIM), jnp.float32).at[:atom_emb.shape[0]].set(atom_emb)
    w1a = emb_lin_W[:, :NODE_DIM]
    w1f = emb_lin_W[:, NODE_DIM:]
    w2h = input_lin_W[:, :HIDDEN_DIM]
    w2t = input_lin_W[:, HIDDEN_DIM:]

    a2, ct, bc = pl.pallas_call(
        _fold_weights_kernel,
        out_shape=(
            jax.ShapeDtypeStruct((128, HIDDEN_DIM), jnp.float32),
            jax.ShapeDtypeStruct((HIDDEN_DIM, FT_DIM), jnp.float32),
            jax.ShapeDtypeStruct((1, HIDDEN_DIM), jnp.float32),
        ),
    )(atom_p, w1a, w1f, w2h, emb_lin_b.reshape(1, -1),
      input_lin_b.reshape(1, -1))

    grid = bn // TILE
    out_flat = pl.pallas_call(
        _main_kernel,
        grid=(grid,),
        in_specs=[
            pl.BlockSpec((TILE, 1), lambda g: (g, 0)),
            pl.BlockSpec((TILE, 1), lambda g: (g, 0)),
            pl.BlockSpec((TILE, FT_DIM), lambda g: (g, 0)),
            pl.BlockSpec((128, HIDDEN_DIM), lambda g: (0, 0)),
            pl.BlockSpec((HIDDEN_DIM, FT_DIM), lambda g: (0, 0)),
            pl.BlockSpec((HIDDEN_DIM, TIME_EMB_DIM), lambda g: (0, 0)),
            pl.BlockSpec((1, HIDDEN_DIM), lambda g: (0, 0)),
        ],
        out_specs=pl.BlockSpec((TILE, HIDDEN_DIM * T_STEPS), lambda g: (g, 0)),
        out_shape=jax.ShapeDtypeStruct((bn, HIDDEN_DIM * T_STEPS),
                                       jnp.float32),
    )(h.reshape(bn, 1).astype(jnp.int32),
      diffusion_t.reshape(bn, 1).astype(jnp.int32),
      f, a2, ct, w2t, bc)

    return out_flat.reshape(bn, HIDDEN_DIM, T_STEPS)


# speedup vs baseline: 4.3611x; 4.3611x over previous
import math

import jax
import jax.numpy as jnp
from jax.experimental import pallas as pl

NODE_DIM = 256
FT_DIM = 256
HIDDEN_DIM = 256
TIME_EMB_DIM = 128
T_STEPS = 8
TILE = 400
MAX_POS = 10000.0


def _fold_weights_kernel(atom_ref, w1_ref, w2_ref, b1_ref, b2_ref,
                         a2_ref, ct_ref, bc_ref):
    w1a = w1_ref[:, :NODE_DIM]
    w1f = w1_ref[:, NODE_DIM:]
    w2h = w2_ref[:, :HIDDEN_DIM]
    atom_p = jnp.concatenate(
        [atom_ref[...],
         jnp.zeros((128 - atom_ref.shape[0], NODE_DIM), jnp.float32)], axis=0)
    a1 = jax.lax.dot_general(atom_p, w1a, (((1,), (1,)), ((), ())),
                             preferred_element_type=jnp.float32)
    a2 = jax.lax.dot_general(a1, w2h, (((1,), (1,)), ((), ())),
                             preferred_element_type=jnp.float32)
    a2_ref[...] = a2.astype(jnp.bfloat16)
    ct_ref[...] = jnp.dot(w2h, w1f,
                          preferred_element_type=jnp.float32).astype(jnp.bfloat16)
    bc_ref[...] = jax.lax.dot_general(b1_ref[...], w2h,
                                      (((1,), (1,)), ((), ())),
                                      preferred_element_type=jnp.float32) + b2_ref[...]


def _main_kernel(h_ref, dt_ref, f_ref, w2_ref, a2_ref, ct_ref, bc_ref,
                 s_ref, out_ref):
    tile = f_ref.shape[0]
    hv = h_ref[:, 0]
    lane = jax.lax.broadcasted_iota(jnp.int32, (tile, 128), 1)
    onehot = (lane == hv[:, None]).astype(jnp.bfloat16)
    o = jnp.dot(onehot, a2_ref[...], preferred_element_type=jnp.float32)
    o += jax.lax.dot_general(f_ref[...].astype(jnp.bfloat16), ct_ref[...],
                             (((1,), (1,)), ((), ())),
                             preferred_element_type=jnp.float32)
    half = TIME_EMB_DIM // 2
    coef = -math.log(MAX_POS) / (half - 1)
    j = jax.lax.broadcasted_iota(jnp.int32, (tile, half), 1).astype(jnp.float32)
    freq = jnp.exp(j * coef)
    args = dt_ref[:, 0].astype(jnp.float32)[:, None] * freq
    temb = jnp.concatenate([jnp.sin(args), jnp.cos(args)], axis=1)
    w2t = w2_ref[:, HIDDEN_DIM:]
    o += jax.lax.dot_general(temb.astype(jnp.bfloat16),
                             w2t.astype(jnp.bfloat16),
                             (((1,), (1,)), ((), ())),
                             preferred_element_type=jnp.float32)
    o += bc_ref[...]
    out_ref[...] = jnp.dot(o.astype(jnp.bfloat16), s_ref[...],
                           preferred_element_type=jnp.float32)


@jax.jit
def kernel(diffusion_t, x, h, f, edge_index, edge_attr, batch, atom_emb,
           emb_lin_W, emb_lin_b, edge_emb_table, input_lin_W, input_lin_b,
           cond_emb_table):
    bn = f.shape[0]
    a2, ct, bc = pl.pallas_call(
        _fold_weights_kernel,
        out_shape=(
            jax.ShapeDtypeStruct((128, HIDDEN_DIM), jnp.bfloat16),
            jax.ShapeDtypeStruct((HIDDEN_DIM, FT_DIM), jnp.bfloat16),
            jax.ShapeDtypeStruct((1, HIDDEN_DIM), jnp.float32),
        ),
    )(atom_emb, emb_lin_W, input_lin_W, emb_lin_b.reshape(1, -1),
      input_lin_b.reshape(1, -1))

    s_mat = (jnp.arange(HIDDEN_DIM * T_STEPS)[None, :] // T_STEPS ==
             jnp.arange(HIDDEN_DIM)[:, None]).astype(jnp.bfloat16)

    grid = bn // TILE
    out_flat = pl.pallas_call(
        _main_kernel,
        grid=(grid,),
        in_specs=[
            pl.BlockSpec((TILE, 1), lambda g: (g, 0)),
            pl.BlockSpec((TILE, 1), lambda g: (g, 0)),
            pl.BlockSpec((TILE, FT_DIM), lambda g: (g, 0)),
            pl.BlockSpec((HIDDEN_DIM, NODE_DIM + TIME_EMB_DIM),
                         lambda g: (0, 0)),
            pl.BlockSpec((128, HIDDEN_DIM), lambda g: (0, 0)),
            pl.BlockSpec((HIDDEN_DIM, FT_DIM), lambda g: (0, 0)),
            pl.BlockSpec((1, HIDDEN_DIM), lambda g: (0, 0)),
            pl.BlockSpec((HIDDEN_DIM, HIDDEN_DIM * T_STEPS),
                         lambda g: (0, 0)),
        ],
        out_specs=pl.BlockSpec((TILE, HIDDEN_DIM * T_STEPS), lambda g: (g, 0)),
        out_shape=jax.ShapeDtypeStruct((bn, HIDDEN_DIM * T_STEPS),
                                       jnp.float32),
    )(h.reshape(bn, 1), diffusion_t.reshape(bn, 1).astype(jnp.int32),
      f, input_lin_W, a2, ct, bc, s_mat)

    return out_flat.reshape(bn, HIDDEN_DIM, T_STEPS)


# speedup vs baseline: 14.9958x; 3.4385x over previous
import math

import jax
import jax.numpy as jnp
from jax.experimental import pallas as pl

NODE_DIM = 256
FT_DIM = 256
HIDDEN_DIM = 256
TIME_EMB_DIM = 128
T_STEPS = 8
TILE = 512
MAX_POS = 10000.0


def _fold_weights_kernel(atom_ref, w1_ref, w2_ref, b1_ref, b2_ref,
                         a2_ref, ct_ref, bc_ref):
    w1a = w1_ref[:, :NODE_DIM]
    w1f = w1_ref[:, NODE_DIM:]
    w2h = w2_ref[:, :HIDDEN_DIM]
    atom_p = jnp.concatenate(
        [atom_ref[...],
         jnp.zeros((128 - atom_ref.shape[0], NODE_DIM), jnp.float32)], axis=0)
    a1 = jax.lax.dot_general(atom_p, w1a, (((1,), (1,)), ((), ())),
                             preferred_element_type=jnp.float32)
    a2 = jax.lax.dot_general(a1, w2h, (((1,), (1,)), ((), ())),
                             preferred_element_type=jnp.float32)
    a2_ref[...] = a2.astype(jnp.bfloat16)
    ct_ref[...] = jnp.dot(w2h, w1f,
                          preferred_element_type=jnp.float32).astype(jnp.bfloat16)
    b1v = b1_ref[...].reshape(1, HIDDEN_DIM)
    b2v = b2_ref[...].reshape(1, HIDDEN_DIM)
    bc_ref[...] = jax.lax.dot_general(b1v, w2h, (((1,), (1,)), ((), ())),
                                      preferred_element_type=jnp.float32) + b2v


def _main_kernel(h_ref, dt_ref, f_ref, w2_ref, a2_ref, ct_ref, bc_ref,
                 out_ref):
    tile = f_ref.shape[0]
    hv = jnp.transpose(h_ref[...].reshape(1, tile), (1, 0))
    lane = jax.lax.broadcasted_iota(jnp.int32, (tile, 128), 1)
    onehot = (lane == hv).astype(jnp.bfloat16)
    o = jnp.dot(onehot, a2_ref[...], preferred_element_type=jnp.float32)
    o += jax.lax.dot_general(f_ref[...].astype(jnp.bfloat16), ct_ref[...],
                             (((1,), (1,)), ((), ())),
                             preferred_element_type=jnp.float32)
    half = TIME_EMB_DIM // 2
    coef = -math.log(MAX_POS) / (half - 1)
    j = jax.lax.broadcasted_iota(jnp.int32, (tile, half), 1).astype(jnp.float32)
    freq = jnp.exp(j * coef)
    dtv = jnp.transpose(dt_ref[...].reshape(1, tile), (1, 0)).astype(jnp.float32)
    args = dtv * freq
    temb = jnp.concatenate([jnp.sin(args), jnp.cos(args)], axis=1)
    w2t = w2_ref[:, HIDDEN_DIM:]
    o += jax.lax.dot_general(temb.astype(jnp.bfloat16),
                             w2t.astype(jnp.bfloat16),
                             (((1,), (1,)), ((), ())),
                             preferred_element_type=jnp.float32)
    out_ref[...] = o + bc_ref[...]


@jax.jit
def kernel(diffusion_t, x, h, f, edge_index, edge_attr, batch, atom_emb,
           emb_lin_W, emb_lin_b, edge_emb_table, input_lin_W, input_lin_b,
           cond_emb_table):
    bn = f.shape[0]
    a2, ct, bc = pl.pallas_call(
        _fold_weights_kernel,
        out_shape=(
            jax.ShapeDtypeStruct((128, HIDDEN_DIM), jnp.bfloat16),
            jax.ShapeDtypeStruct((HIDDEN_DIM, FT_DIM), jnp.bfloat16),
            jax.ShapeDtypeStruct((1, HIDDEN_DIM), jnp.float32),
        ),
    )(atom_emb, emb_lin_W, input_lin_W, emb_lin_b, input_lin_b)

    grid = pl.cdiv(bn, TILE)
    out_flat = pl.pallas_call(
        _main_kernel,
        grid=(grid,),
        in_specs=[
            pl.BlockSpec((TILE,), lambda g: (g,)),
            pl.BlockSpec((TILE,), lambda g: (g,)),
            pl.BlockSpec((TILE, FT_DIM), lambda g: (g, 0)),
            pl.BlockSpec((HIDDEN_DIM, NODE_DIM + TIME_EMB_DIM),
                         lambda g: (0, 0)),
            pl.BlockSpec((128, HIDDEN_DIM), lambda g: (0, 0)),
            pl.BlockSpec((HIDDEN_DIM, FT_DIM), lambda g: (0, 0)),
            pl.BlockSpec((1, HIDDEN_DIM), lambda g: (0, 0)),
        ],
        out_specs=pl.BlockSpec((TILE, HIDDEN_DIM), lambda g: (g, 0)),
        out_shape=jax.ShapeDtypeStruct((bn, HIDDEN_DIM), jnp.float32),
    )(h, diffusion_t.astype(jnp.int32), f, input_lin_W, a2, ct, bc)

    return jnp.broadcast_to(out_flat[:, :, None], (bn, HIDDEN_DIM, T_STEPS))


# speedup vs baseline: 21.0474x; 1.4036x over previous
import math

import jax
import jax.numpy as jnp
from jax.experimental import pallas as pl

NODE_DIM = 256
FT_DIM = 256
HIDDEN_DIM = 256
TIME_EMB_DIM = 128
T_STEPS = 8
TILE = 512
MAX_POS = 10000.0


def _fold_weights_kernel(atom_ref, w1_ref, w2_ref, b1_ref, b2_ref,
                         a2_ref, ct_ref, bc_ref):
    w1a = w1_ref[:, :NODE_DIM]
    w1f = w1_ref[:, NODE_DIM:]
    w2h = w2_ref[:, :HIDDEN_DIM]
    atom_p = jnp.concatenate(
        [atom_ref[...],
         jnp.zeros((128 - atom_ref.shape[0], NODE_DIM), jnp.float32)], axis=0)
    a1 = jax.lax.dot_general(atom_p, w1a, (((1,), (1,)), ((), ())),
                             preferred_element_type=jnp.float32)
    a2 = jax.lax.dot_general(a1, w2h, (((1,), (1,)), ((), ())),
                             preferred_element_type=jnp.float32)
    a2_ref[...] = a2.astype(jnp.bfloat16)
    ct_ref[...] = jnp.dot(w2h, w1f,
                          preferred_element_type=jnp.float32).astype(jnp.bfloat16)
    b1v = b1_ref[...].reshape(1, HIDDEN_DIM)
    b2v = b2_ref[...].reshape(1, HIDDEN_DIM)
    bc_ref[...] = jax.lax.dot_general(b1v, w2h, (((1,), (1,)), ((), ())),
                                      preferred_element_type=jnp.float32) + b2v


def _main_kernel(h_ref, dt_ref, f_ref, w2_ref, a2_ref, ct_ref, bc_ref,
                 out_ref):
    tile = f_ref.shape[0]
    hv = jnp.transpose(h_ref[...].reshape(1, tile), (1, 0))
    lane = jax.lax.broadcasted_iota(jnp.int32, (tile, 128), 1)
    onehot = (lane == hv).astype(jnp.bfloat16)
    o = jnp.dot(onehot, a2_ref[...], preferred_element_type=jnp.float32)
    o += jax.lax.dot_general(f_ref[...].astype(jnp.bfloat16), ct_ref[...],
                             (((1,), (1,)), ((), ())),
                             preferred_element_type=jnp.float32)
    half = TIME_EMB_DIM // 2
    coef = -math.log(MAX_POS) / (half - 1)
    j = jax.lax.broadcasted_iota(jnp.int32, (tile, half), 1).astype(jnp.float32)
    freq = jnp.exp(j * coef)
    dtv = jnp.transpose(dt_ref[...].reshape(1, tile), (1, 0)).astype(jnp.float32)
    args = dtv * freq
    temb = jnp.concatenate([jnp.sin(args), jnp.cos(args)], axis=1)
    w2t = w2_ref[:, HIDDEN_DIM:]
    o += jax.lax.dot_general(temb.astype(jnp.bfloat16),
                             w2t.astype(jnp.bfloat16),
                             (((1,), (1,)), ((), ())),
                             preferred_element_type=jnp.float32)
    o += bc_ref[...]
    out_ref[...] = jnp.broadcast_to(o[:, None, :], (tile, T_STEPS, HIDDEN_DIM))


@jax.jit
def kernel(diffusion_t, x, h, f, edge_index, edge_attr, batch, atom_emb,
           emb_lin_W, emb_lin_b, edge_emb_table, input_lin_W, input_lin_b,
           cond_emb_table):
    bn = f.shape[0]
    a2, ct, bc = pl.pallas_call(
        _fold_weights_kernel,
        out_shape=(
            jax.ShapeDtypeStruct((128, HIDDEN_DIM), jnp.bfloat16),
            jax.ShapeDtypeStruct((HIDDEN_DIM, FT_DIM), jnp.bfloat16),
            jax.ShapeDtypeStruct((1, HIDDEN_DIM), jnp.float32),
        ),
    )(atom_emb, emb_lin_W, input_lin_W, emb_lin_b, input_lin_b)

    grid = pl.cdiv(bn, TILE)
    out_flat = pl.pallas_call(
        _main_kernel,
        grid=(grid,),
        in_specs=[
            pl.BlockSpec((TILE,), lambda g: (g,)),
            pl.BlockSpec((TILE,), lambda g: (g,)),
            pl.BlockSpec((TILE, FT_DIM), lambda g: (g, 0)),
            pl.BlockSpec((HIDDEN_DIM, NODE_DIM + TIME_EMB_DIM),
                         lambda g: (0, 0)),
            pl.BlockSpec((128, HIDDEN_DIM), lambda g: (0, 0)),
            pl.BlockSpec((HIDDEN_DIM, FT_DIM), lambda g: (0, 0)),
            pl.BlockSpec((1, HIDDEN_DIM), lambda g: (0, 0)),
        ],
        out_specs=pl.BlockSpec((TILE, T_STEPS, HIDDEN_DIM), lambda g: (g, 0, 0)),
        out_shape=jax.ShapeDtypeStruct((bn, T_STEPS, HIDDEN_DIM), jnp.float32),
    )(h, diffusion_t.astype(jnp.int32), f, input_lin_W, a2, ct, bc)

    return jnp.transpose(out_flat, (0, 2, 1))
